# baseline (speedup 1.0000x reference)
import functools

import jax
import jax.numpy as jnp
from jax import lax
from jax.experimental import pallas as pl
from jax.experimental.pallas import tpu as pltpu
from jax.experimental.pallas import tpu_sc as plsc

_EPS = 1e-8
_KIN_COEF = 0.1
_TOPK = 4
_TILE = 4096


def _score_topk_body(K, nt, q_ref, k_ref, ml_ref, L_ref, outs_ref, outi_ref,
                     qn_ref, t0, t1, t2, t3, i0, i1, i2, i3):
    B = q_ref.shape[0]
    D = q_ref.shape[1]
    t = pl.program_id(0)

    @pl.when(t == 0)
    def _init():
        q = q_ref[...]
        qn_ref[...] = q * (1.0 / (jnp.sqrt(jnp.sum(q * q, axis=1, keepdims=True)) + _EPS))
        neg = jnp.full((B, 1), -jnp.inf, jnp.float32)
        z = jnp.zeros((B, 1), jnp.int32)
        for ref in (t0, t1, t2, t3):
            ref[...] = neg
        for ref in (i0, i1, i2, i3):
            ref[...] = z

    keys = k_ref[...]
    norm2 = jnp.sum(keys * keys, axis=1, keepdims=True)
    kn = keys * (1.0 / (jnp.sqrt(norm2) + _EPS))
    s = lax.dot_general(qn_ref[...], kn, (((1,), (1,)), ((), ())),
                        preferred_element_type=jnp.float32)

    ml = jnp.maximum(ml_ref[...].astype(jnp.float32), 1.0)
    L = jnp.maximum(L_ref[...].astype(jnp.float32), 1.0)
    rel = jnp.abs(ml - L) / jnp.maximum(ml, L)
    kin = jnp.exp(rel * (-_KIN_COEF))

    col = lax.broadcasted_iota(jnp.int32, s.shape, 1)
    valid = (col + t * _TILE) < K
    s = jnp.where(valid, s * kin, -jnp.inf)

    Ts = [t0[...], t1[...], t2[...], t3[...]]
    Is = [i0[...], i1[...], i2[...], i3[...]]
    for r in range(_TOPK):
        m = jnp.max(s, axis=1, keepdims=True)
        hit = s == m
        a = jnp.min(jnp.where(hit, col, _TILE), axis=1, keepdims=True)
        gidx = a + t * _TILE
        if r < _TOPK - 1:
            s = jnp.where(col == a, -jnp.inf, s)
        c = [m > Ts[j] for j in range(_TOPK)]
        nT = [None] * _TOPK
        nI = [None] * _TOPK
        nT[0] = jnp.where(c[0], m, Ts[0])
        nI[0] = jnp.where(c[0], gidx, Is[0])
        for j in range(1, _TOPK):
            nT[j] = jnp.where(c[j], jnp.where(c[j - 1], Ts[j - 1], m), Ts[j])
            nI[j] = jnp.where(c[j], jnp.where(c[j - 1], Is[j - 1], gidx), Is[j])
        Ts, Is = nT, nI
    t0[...], t1[...], t2[...], t3[...] = Ts
    i0[...], i1[...], i2[...], i3[...] = Is

    @pl.when(t == nt - 1)
    def _emit():
        outs_ref[...] = jnp.concatenate(Ts, axis=1)
        outi_ref[...] = jnp.concatenate(Is, axis=1)


def _score_topk(qf, keys, mlf, Lf):
    B, D = qf.shape
    K = keys.shape[0]
    nt = pl.cdiv(K, _TILE)
    body = functools.partial(_score_topk_body, K, nt)
    return pl.pallas_call(
        body,
        grid=(nt,),
        in_specs=[
            pl.BlockSpec((B, D), lambda t: (0, 0)),
            pl.BlockSpec((_TILE, D), lambda t: (t, 0)),
            pl.BlockSpec((1, _TILE), lambda t: (0, t)),
            pl.BlockSpec((B, 1), lambda t: (0, 0)),
        ],
        out_specs=[
            pl.BlockSpec((B, _TOPK), lambda t: (0, 0)),
            pl.BlockSpec((B, _TOPK), lambda t: (0, 0)),
        ],
        out_shape=[
            jax.ShapeDtypeStruct((B, _TOPK), jnp.float32),
            jax.ShapeDtypeStruct((B, _TOPK), jnp.int32),
        ],
        scratch_shapes=(
            [pltpu.VMEM((B, D), jnp.float32)]
            + [pltpu.VMEM((B, 1), jnp.float32) for _ in range(_TOPK)]
            + [pltpu.VMEM((B, 1), jnp.int32) for _ in range(_TOPK)]
        ),
        compiler_params=pltpu.CompilerParams(
            dimension_semantics=("arbitrary",)),
    )(qf, keys, mlf, Lf)


def _sc_gather(table, idx_flat):
    R = idx_flat.shape[0]
    D = table.shape[1]
    info = plsc.get_sparse_core_info()
    NC, NS = info.num_cores, info.num_subcores
    NW = NC * NS
    b_per_w = R // NW
    mesh = plsc.VectorSubcoreMesh(core_axis_name="c", subcore_axis_name="s")

    @functools.partial(
        pl.kernel, mesh=mesh,
        out_type=jax.ShapeDtypeStruct((R, D), jnp.float32),
        scratch_types=[
            pltpu.VMEM((b_per_w,), jnp.int32),
            pltpu.VMEM((b_per_w, D), jnp.float32),
            pltpu.SemaphoreType.DMA,
        ],
    )
    def gather_k(table_hbm, idx_hbm, out_hbm, idx_v, rows_v, sem):
        wid = lax.axis_index("s") * NC + lax.axis_index("c")
        base = wid * b_per_w
        pltpu.sync_copy(idx_hbm.at[pl.ds(base, b_per_w)], idx_v)
        pltpu.async_copy(table_hbm.at[idx_v], rows_v, sem).wait()
        pltpu.sync_copy(rows_v, out_hbm.at[pl.ds(base, b_per_w)])

    return gather_k(table, idx_flat)


def _proj_body(g_ref, w_ref, b_ref, o_ref):
    o_ref[...] = (
        jnp.dot(g_ref[...], w_ref[...], preferred_element_type=jnp.float32)
        + b_ref[...]
    )


def _proj(g, W, b):
    R, D = g.shape
    return pl.pallas_call(
        _proj_body,
        out_shape=jax.ShapeDtypeStruct((R, D), jnp.float32),
    )(g, W, b.reshape(1, D))


def kernel(query_features, text_features, m_lengths, lengths, W_proj, b_proj):
    B, D = query_features.shape
    K = text_features.shape[0]
    nt = pl.cdiv(K, _TILE)

    mlf = m_lengths.reshape(1, K)
    Lf = lengths.reshape(B, 1)

    top_scores, top_idx = _score_topk(query_features, text_features, mlf, Lf)
    gathered = _sc_gather(text_features, top_idx.reshape(B * _TOPK))
    re_feat = _proj(gathered, W_proj, b_proj).reshape(B, _TOPK, D)
    return top_scores, top_idx, re_feat


# speedup vs baseline: 1.0066x; 1.0066x over previous
import functools

import jax
import jax.numpy as jnp
from jax import lax
from jax.experimental import pallas as pl
from jax.experimental.pallas import tpu as pltpu
from jax.experimental.pallas import tpu_sc as plsc

_EPS = 1e-8
_KIN_COEF = 0.1
_TOPK = 4
_TILE = 4096


def _score_topk_body(K, nt, q_ref, k_ref, ml_ref, L_ref, outs_ref, outi_ref,
                     qn_ref, t0, t1, t2, t3, i0, i1, i2, i3):
    B = q_ref.shape[0]
    D = q_ref.shape[1]
    t = pl.program_id(0)

    @pl.when(t == 0)
    def _init():
        q = q_ref[...]
        qn_ref[...] = q * (1.0 / (jnp.sqrt(jnp.sum(q * q, axis=1, keepdims=True)) + _EPS))
        neg = jnp.full((B, 1), -jnp.inf, jnp.float32)
        z = jnp.zeros((B, 1), jnp.int32)
        for ref in (t0, t1, t2, t3):
            ref[...] = neg
        for ref in (i0, i1, i2, i3):
            ref[...] = z

    keys = k_ref[...]
    norm2 = jnp.sum(keys * keys, axis=1, keepdims=True)
    kn = keys * (1.0 / (jnp.sqrt(norm2) + _EPS))
    s = lax.dot_general(qn_ref[...], kn, (((1,), (1,)), ((), ())),
                        preferred_element_type=jnp.float32)

    ml = ml_ref[...]
    L = L_ref[...]
    rel = jnp.abs(ml - L) / jnp.maximum(ml, L)
    kin = jnp.exp(rel * (-_KIN_COEF))

    col = lax.broadcasted_iota(jnp.int32, s.shape, 1)
    valid = (col + t * _TILE) < K
    s = jnp.where(valid, s * kin, -jnp.inf)

    Ts = [t0[...], t1[...], t2[...], t3[...]]
    Is = [i0[...], i1[...], i2[...], i3[...]]
    for r in range(_TOPK):
        m = jnp.max(s, axis=1, keepdims=True)
        hit = s == m
        a = jnp.min(jnp.where(hit, col, _TILE), axis=1, keepdims=True)
        gidx = a + t * _TILE
        if r < _TOPK - 1:
            s = jnp.where(col == a, -jnp.inf, s)
        c = [m > Ts[j] for j in range(_TOPK)]
        nT = [None] * _TOPK
        nI = [None] * _TOPK
        nT[0] = jnp.where(c[0], m, Ts[0])
        nI[0] = jnp.where(c[0], gidx, Is[0])
        for j in range(1, _TOPK):
            nT[j] = jnp.where(c[j], jnp.where(c[j - 1], Ts[j - 1], m), Ts[j])
            nI[j] = jnp.where(c[j], jnp.where(c[j - 1], Is[j - 1], gidx), Is[j])
        Ts, Is = nT, nI
    t0[...], t1[...], t2[...], t3[...] = Ts
    i0[...], i1[...], i2[...], i3[...] = Is

    @pl.when(t == nt - 1)
    def _emit():
        outs_ref[...] = jnp.concatenate(Ts, axis=1)
        outi_ref[...] = jnp.concatenate(Is, axis=1)


def _score_topk(qf, keys, mlf, Lf):
    B, D = qf.shape
    K = keys.shape[0]
    nt = pl.cdiv(K, _TILE)
    body = functools.partial(_score_topk_body, K, nt)
    return pl.pallas_call(
        body,
        grid=(nt,),
        in_specs=[
            pl.BlockSpec((B, D), lambda t: (0, 0)),
            pl.BlockSpec((_TILE, D), lambda t: (t, 0)),
            pl.BlockSpec((1, _TILE), lambda t: (0, t)),
            pl.BlockSpec((B, 1), lambda t: (0, 0)),
        ],
        out_specs=[
            pl.BlockSpec((B, _TOPK), lambda t: (0, 0)),
            pl.BlockSpec((B, _TOPK), lambda t: (0, 0)),
        ],
        out_shape=[
            jax.ShapeDtypeStruct((B, _TOPK), jnp.float32),
            jax.ShapeDtypeStruct((B, _TOPK), jnp.int32),
        ],
        scratch_shapes=(
            [pltpu.VMEM((B, D), jnp.float32)]
            + [pltpu.VMEM((B, 1), jnp.float32) for _ in range(_TOPK)]
            + [pltpu.VMEM((B, 1), jnp.int32) for _ in range(_TOPK)]
        ),
        compiler_params=pltpu.CompilerParams(
            dimension_semantics=("arbitrary",)),
    )(qf, keys, mlf, Lf)


def _sc_gather(table, idx_flat):
    R = idx_flat.shape[0]
    D = table.shape[1]
    info = plsc.get_sparse_core_info()
    NC, NS = info.num_cores, info.num_subcores
    NW = NC * NS
    b_per_w = R // NW
    mesh = plsc.VectorSubcoreMesh(core_axis_name="c", subcore_axis_name="s")

    @functools.partial(
        pl.kernel, mesh=mesh,
        out_type=jax.ShapeDtypeStruct((R, D), jnp.float32),
        scratch_types=[
            pltpu.VMEM((b_per_w,), jnp.int32),
            pltpu.VMEM((b_per_w, D), jnp.float32),
            pltpu.SemaphoreType.DMA,
        ],
    )
    def gather_k(table_hbm, idx_hbm, out_hbm, idx_v, rows_v, sem):
        wid = lax.axis_index("s") * NC + lax.axis_index("c")
        base = wid * b_per_w
        pltpu.sync_copy(idx_hbm.at[pl.ds(base, b_per_w)], idx_v)
        pltpu.async_copy(table_hbm.at[idx_v], rows_v, sem).wait()
        pltpu.sync_copy(rows_v, out_hbm.at[pl.ds(base, b_per_w)])

    return gather_k(table, idx_flat)


def _proj_body(g_ref, w_ref, b_ref, o_ref):
    o_ref[...] = (
        jnp.dot(g_ref[...], w_ref[...], preferred_element_type=jnp.float32)
        + b_ref[...]
    )


def _proj(g, W, b):
    R, D = g.shape
    return pl.pallas_call(
        _proj_body,
        out_shape=jax.ShapeDtypeStruct((R, D), jnp.float32),
    )(g, W, b.reshape(1, D))


def kernel(query_features, text_features, m_lengths, lengths, W_proj, b_proj):
    B, D = query_features.shape
    K = text_features.shape[0]
    nt = pl.cdiv(K, _TILE)

    mlf = jnp.maximum(m_lengths.astype(jnp.float32), 1.0)
    mlf = jnp.pad(mlf, (0, nt * _TILE - K), constant_values=1.0).reshape(1, nt * _TILE)
    Lf = jnp.maximum(lengths.astype(jnp.float32), 1.0).reshape(B, 1)

    top_scores, top_idx = _score_topk(query_features, text_features, mlf, Lf)
    gathered = _sc_gather(text_features, top_idx.reshape(B * _TOPK))
    re_feat = _proj(gathered, W_proj, b_proj).reshape(B, _TOPK, D)
    return top_scores, top_idx, re_feat
